# baseline (speedup 1.0000x reference)
import functools

import jax
import jax.numpy as jnp
from jax import lax
from jax.experimental import pallas as pl
from jax.experimental.pallas import tpu as pltpu
from jax.experimental.pallas import tpu_sc as plsc

N_L = 500
C = 8
B = 4
L = 256
D_ENC = 768
D_TYPE = 256
D_MODEL = 1024
E_LBL = 8000

NC, NS, LANES = 2, 16, 16
NW = NC * NS
GE = D_ENC // LANES
E_PAD = 8192
N_PAD = 512
EBLK = 1024
TCHUNK = 16
NCH = L // TCHUNK


def _sc_body(ids_hbm, tok_hbm, ts_out, idx_v, buf0, buf1, acc_v, sem0, sem1):
    c = lax.axis_index("c")
    s = lax.axis_index("s")
    wid = c * NS + s
    zero16 = jnp.zeros((LANES,), jnp.float32)

    pltpu.sync_copy(ids_hbm.at[wid], idx_v)
    pltpu.async_copy(tok_hbm.at[idx_v.at[pl.ds(0, TCHUNK)]], buf0, sem0)
    pltpu.async_copy(tok_hbm.at[idx_v.at[pl.ds(TCHUNK, TCHUNK)]], buf1, sem1)
    accs = tuple(zero16 for _ in range(GE))

    def pair_body(i, accs):
        for half, (buf, sem) in enumerate(((buf0, sem0), (buf1, sem1))):
            pltpu.make_async_copy(
                tok_hbm.at[idx_v.at[pl.ds(0, TCHUNK)]], buf, sem).wait()
            def rbody(r, accs, buf=buf):
                return tuple(accs[g] + buf[r, pl.ds(g * LANES, LANES)]
                             for g in range(GE))
            accs = lax.fori_loop(0, TCHUNK, rbody, accs)

            @pl.when(i < NCH // 2 - 1)
            def _(buf=buf, sem=sem, half=half):
                nxt = (2 * i + 2 + half) * TCHUNK
                pltpu.async_copy(
                    tok_hbm.at[idx_v.at[pl.ds(nxt, TCHUNK)]], buf, sem)
        return accs

    accs = lax.fori_loop(0, NCH // 2, pair_body, accs)
    for g in range(GE):
        acc_v[pl.ds(g * LANES, LANES)] = accs[g]
    pltpu.sync_copy(acc_v, ts_out.at[wid])


@functools.cache
def _get_sc_call():
    return functools.partial(
        pl.kernel,
        out_type=jax.ShapeDtypeStruct((NW, D_ENC), jnp.float32),
        mesh=plsc.VectorSubcoreMesh(core_axis_name="c", subcore_axis_name="s",
                                    num_cores=NC, num_subcores=NS),
        scratch_types=[
            pltpu.VMEM((L,), jnp.int32),
            pltpu.VMEM((TCHUNK, D_ENC), jnp.float32),
            pltpu.VMEM((TCHUNK, D_ENC), jnp.float32),
            pltpu.VMEM((D_ENC,), jnp.float32),
            pltpu.SemaphoreType.DMA,
            pltpu.SemaphoreType.DMA,
        ],
    )(_sc_body)


def _tca_body(edges_ref, lraw_ref, nte_ref, w_ref, a_out, inv_out, misc_out,
              wtop_out):
    cmat = jnp.zeros((N_PAD, N_PAD), jnp.float32)
    for k in range(E_PAD // EBLK):
        src = edges_ref[0, pl.ds(k * EBLK, EBLK)]
        dst = edges_ref[1, pl.ds(k * EBLK, EBLK)]
        node_iota = lax.broadcasted_iota(jnp.int32, (EBLK, N_PAD), 1)
        one = jnp.float32(1.0)
        s_src = jnp.where(src[:, None] == node_iota, one, 0.0).astype(
            jnp.bfloat16)
        s_dst = jnp.where(dst[:, None] == node_iota, one, 0.0).astype(
            jnp.bfloat16)
        cmat = cmat + lax.dot_general(
            s_dst, s_src, (((0,), (0,)), ((), ())),
            preferred_element_type=jnp.float32)
    lraw = lraw_ref[...]
    rawp = jnp.pad(lraw, ((0, N_PAD - N_L), (0, 0)))
    llraw = jnp.dot(cmat, rawp, preferred_element_type=jnp.float32)
    cnt = cmat.sum(axis=1)
    le = nte_ref[0]
    ne = nte_ref[1]
    inv = 1.0 / (cnt + float(C))
    lln = jnp.concatenate(
        [llraw * inv[:, None], (cnt * inv)[:, None] * le[None, :]], axis=1)
    a_out[...] = jnp.dot(lln, w_ref[...],
                         preferred_element_type=jnp.float32
                         ).astype(jnp.bfloat16)
    inv_out[...] = inv[:, None]
    s_label = jnp.concatenate([jnp.sum(lraw, axis=0), float(N_L) * le])
    sw = jnp.dot((s_label * (1.0 / (N_L + 1)))[None, :], w_ref[...],
                 preferred_element_type=jnp.float32)
    new_ = jnp.dot(ne[None, :], w_ref[D_ENC:, :],
                   preferred_element_type=jnp.float32)
    misc_out[...] = jnp.concatenate([sw, new_], axis=0)
    wtop_out[...] = w_ref[:D_ENC, :].astype(jnp.bfloat16)


_tca_call = pl.pallas_call(
    _tca_body,
    out_shape=[
        jax.ShapeDtypeStruct((N_PAD, D_MODEL), jnp.bfloat16),
        jax.ShapeDtypeStruct((N_PAD, 1), jnp.float32),
        jax.ShapeDtypeStruct((2, D_MODEL), jnp.float32),
        jax.ShapeDtypeStruct((D_ENC, D_MODEL), jnp.bfloat16),
    ],
)


def _tcb_body(ts_ref, a_ref, inv_ref, misc_ref, wtop_ref, out_ref):
    te = ts_ref[...] * (1.0 / float(L))
    tew = jnp.dot(te.astype(jnp.bfloat16), wtop_ref[...],
                  preferred_element_type=jnp.float32)
    sw = misc_ref[0]
    new_ = misc_ref[1]
    a = a_ref[...].astype(jnp.float32)
    inv = inv_ref[...]
    tew3 = tew.reshape(B, C, D_MODEL)
    u_w = tew3.sum(axis=1)
    for b in range(B):
        t_b = u_w[b] + float(C) * new_
        xl = jnp.maximum(a + inv * t_b[None, :], 0.0)
        out_ref[b * (N_L + C):b * (N_L + C) + N_L, :] = xl[:N_L]
        xt = jnp.maximum(
            sw[None, :] + (tew3[b] + new_[None, :]) * (1.0 / (N_L + 1)), 0.0)
        out_ref[b * (N_L + C) + N_L:(b + 1) * (N_L + C), :] = xt


_tcb_call = pl.pallas_call(
    _tcb_body,
    out_shape=jax.ShapeDtypeStruct((B * (N_L + C), D_MODEL), jnp.float32),
)


def kernel(input_ids, attention_mask, nchunks, label_edges, tok_embed,
           node_type_embeddings, label_nodes_raw, W):
    ids = input_ids.reshape(NW, L).astype(jnp.int32)
    pad = E_PAD - E_LBL
    src = jnp.concatenate(
        [label_edges[0].astype(jnp.int32), jnp.zeros((pad,), jnp.int32)])
    dst = jnp.concatenate(
        [label_edges[1].astype(jnp.int32),
         jnp.full((pad,), N_PAD - 1, jnp.int32)])
    edges = jnp.stack([src, dst])
    ts = _get_sc_call()(ids, tok_embed)
    a, inv, misc, wtop = _tca_call(
        edges, label_nodes_raw, node_type_embeddings, W)
    return _tcb_call(ts, a, inv, misc, wtop)


# speedup vs baseline: 1.0372x; 1.0372x over previous
import functools

import jax
import jax.numpy as jnp
from jax import lax
from jax.experimental import pallas as pl
from jax.experimental.pallas import tpu as pltpu
from jax.experimental.pallas import tpu_sc as plsc

N_L = 500
C = 8
B = 4
L = 256
D_ENC = 768
D_TYPE = 256
D_MODEL = 1024
E_LBL = 8000

NC, NS, LANES = 2, 16, 16
NW = NC * NS
GE = D_ENC // LANES
E_PAD = 8192
N_PAD = 512
EBLK = 1024
TCHUNK = 32
NCH = L // TCHUNK


def _sc_body(ids_hbm, tok_hbm, ts_out, idx_v, buf0, buf1, acc_v, sem0, sem1):
    c = lax.axis_index("c")
    s = lax.axis_index("s")
    wid = c * NS + s
    zero16 = jnp.zeros((LANES,), jnp.float32)

    pltpu.sync_copy(ids_hbm.at[wid], idx_v)
    pltpu.async_copy(tok_hbm.at[idx_v.at[pl.ds(0, TCHUNK)]], buf0, sem0)
    pltpu.async_copy(tok_hbm.at[idx_v.at[pl.ds(TCHUNK, TCHUNK)]], buf1, sem1)
    accs = tuple(zero16 for _ in range(GE))

    def pair_body(i, accs):
        for half, (buf, sem) in enumerate(((buf0, sem0), (buf1, sem1))):
            pltpu.make_async_copy(
                tok_hbm.at[idx_v.at[pl.ds(0, TCHUNK)]], buf, sem).wait()
            def rbody(r, accs, buf=buf):
                return tuple(accs[g] + buf[r, pl.ds(g * LANES, LANES)]
                             for g in range(GE))
            accs = lax.fori_loop(0, TCHUNK, rbody, accs)

            @pl.when(i < NCH // 2 - 1)
            def _(buf=buf, sem=sem, half=half):
                nxt = (2 * i + 2 + half) * TCHUNK
                pltpu.async_copy(
                    tok_hbm.at[idx_v.at[pl.ds(nxt, TCHUNK)]], buf, sem)
        return accs

    accs = lax.fori_loop(0, NCH // 2, pair_body, accs)
    for g in range(GE):
        acc_v[pl.ds(g * LANES, LANES)] = accs[g]
    pltpu.sync_copy(acc_v, ts_out.at[wid])


@functools.cache
def _get_sc_call():
    return functools.partial(
        pl.kernel,
        out_type=jax.ShapeDtypeStruct((NW, D_ENC), jnp.float32),
        mesh=plsc.VectorSubcoreMesh(core_axis_name="c", subcore_axis_name="s",
                                    num_cores=NC, num_subcores=NS),
        scratch_types=[
            pltpu.VMEM((L,), jnp.int32),
            pltpu.VMEM((TCHUNK, D_ENC), jnp.float32),
            pltpu.VMEM((TCHUNK, D_ENC), jnp.float32),
            pltpu.VMEM((D_ENC,), jnp.float32),
            pltpu.SemaphoreType.DMA,
            pltpu.SemaphoreType.DMA,
        ],
    )(_sc_body)


def _tca_body(edges_ref, lraw_ref, nte_ref, w_ref, a_out, inv_out, misc_out,
              wtop_out):
    cmat = jnp.zeros((N_PAD, N_PAD), jnp.float32)
    for k in range(E_PAD // EBLK):
        src = edges_ref[0, pl.ds(k * EBLK, EBLK)]
        dst = edges_ref[1, pl.ds(k * EBLK, EBLK)]
        node_iota = lax.broadcasted_iota(jnp.int32, (EBLK, N_PAD), 1)
        one = jnp.float32(1.0)
        s_src = jnp.where(src[:, None] == node_iota, one, 0.0).astype(
            jnp.bfloat16)
        s_dst = jnp.where(dst[:, None] == node_iota, one, 0.0).astype(
            jnp.bfloat16)
        cmat = cmat + lax.dot_general(
            s_dst, s_src, (((0,), (0,)), ((), ())),
            preferred_element_type=jnp.float32)
    lraw = lraw_ref[...]
    rawp = jnp.pad(lraw, ((0, N_PAD - N_L), (0, 0)))
    llraw = jnp.dot(cmat, rawp, preferred_element_type=jnp.float32)
    cnt = cmat.sum(axis=1)
    le = nte_ref[0]
    ne = nte_ref[1]
    inv = 1.0 / (cnt + float(C))
    lln = jnp.concatenate(
        [llraw * inv[:, None], (cnt * inv)[:, None] * le[None, :]], axis=1)
    a_out[...] = jnp.dot(lln, w_ref[...],
                         preferred_element_type=jnp.float32
                         ).astype(jnp.bfloat16)
    inv_out[...] = inv[:, None]
    s_label = jnp.concatenate([jnp.sum(lraw, axis=0), float(N_L) * le])
    sw = jnp.dot((s_label * (1.0 / (N_L + 1)))[None, :], w_ref[...],
                 preferred_element_type=jnp.float32)
    new_ = jnp.dot(ne[None, :], w_ref[D_ENC:, :],
                   preferred_element_type=jnp.float32)
    misc_out[...] = jnp.concatenate([sw, new_], axis=0)
    wtop_out[...] = w_ref[:D_ENC, :].astype(jnp.bfloat16)


_tca_call = pl.pallas_call(
    _tca_body,
    out_shape=[
        jax.ShapeDtypeStruct((N_PAD, D_MODEL), jnp.bfloat16),
        jax.ShapeDtypeStruct((N_PAD, 1), jnp.float32),
        jax.ShapeDtypeStruct((2, D_MODEL), jnp.float32),
        jax.ShapeDtypeStruct((D_ENC, D_MODEL), jnp.bfloat16),
    ],
)


def _tcb_body(ts_ref, a_ref, inv_ref, misc_ref, wtop_ref, out_ref):
    te = ts_ref[...] * (1.0 / float(L))
    tew = jnp.dot(te.astype(jnp.bfloat16), wtop_ref[...],
                  preferred_element_type=jnp.float32)
    sw = misc_ref[0]
    new_ = misc_ref[1]
    a = a_ref[...].astype(jnp.float32)
    inv = inv_ref[...]
    tew3 = tew.reshape(B, C, D_MODEL)
    u_w = tew3.sum(axis=1)
    for b in range(B):
        t_b = u_w[b] + float(C) * new_
        xl = jnp.maximum(a + inv * t_b[None, :], 0.0)
        out_ref[b * (N_L + C):b * (N_L + C) + N_L, :] = xl[:N_L]
        xt = jnp.maximum(
            sw[None, :] + (tew3[b] + new_[None, :]) * (1.0 / (N_L + 1)), 0.0)
        out_ref[b * (N_L + C) + N_L:(b + 1) * (N_L + C), :] = xt


_tcb_call = pl.pallas_call(
    _tcb_body,
    out_shape=jax.ShapeDtypeStruct((B * (N_L + C), D_MODEL), jnp.float32),
)


def kernel(input_ids, attention_mask, nchunks, label_edges, tok_embed,
           node_type_embeddings, label_nodes_raw, W):
    ids = input_ids.reshape(NW, L).astype(jnp.int32)
    pad = E_PAD - E_LBL
    src = jnp.concatenate(
        [label_edges[0].astype(jnp.int32), jnp.zeros((pad,), jnp.int32)])
    dst = jnp.concatenate(
        [label_edges[1].astype(jnp.int32),
         jnp.full((pad,), N_PAD - 1, jnp.int32)])
    edges = jnp.stack([src, dst])
    ts = _get_sc_call()(ids, tok_embed)
    a, inv, misc, wtop = _tca_call(
        edges, label_nodes_raw, node_type_embeddings, W)
    return _tcb_call(ts, a, inv, misc, wtop)
